# baseline (speedup 1.0000x reference)
import functools

import jax
import jax.numpy as jnp
from jax import lax
from jax.experimental import pallas as pl
from jax.experimental.pallas import tpu as pltpu
from jax.experimental.pallas import tpu_sc as plsc

TILE = 256
K = 2
NW = 32
LANES = 16


def _router_body(T, E, NT, NRT, x_ref, wg_ref, dst_ref, wts_ref, eot_ref):
    f32 = jnp.float32
    x = x_ref[...]
    wg = wg_ref[...]
    scores = jax.nn.sigmoid(
        lax.dot_general(x, wg, (((1,), (1,)), ((), ())),
                        preferred_element_type=f32))

    U8 = (lax.broadcasted_iota(jnp.int32, (E, E), 0)
          <= lax.broadcasted_iota(jnp.int32, (E, E), 1)).astype(f32)

    def first_max_onehot(s):
        m = jnp.max(s, axis=1, keepdims=True)
        eq = (s == m).astype(f32)
        c = lax.dot_general(eq, U8, (((1,), (0,)), ((), ())),
                            preferred_element_type=f32)
        return eq * (c == 1.0).astype(f32), m

    oh1, m1 = first_max_onehot(scores)
    oh2, m2 = first_max_onehot(scores - oh1 * 1e9)
    ssum = m1 + m2
    wts_ref[...] = jnp.concatenate([m1 / ssum, m2 / ssum], axis=1)

    mask = oh1 + oh2

    Lt = (lax.broadcasted_iota(jnp.int32, (T, T), 0)
          > lax.broadcasted_iota(jnp.int32, (T, T), 1)).astype(jnp.bfloat16)
    pos = lax.dot_general(Lt, mask.astype(jnp.bfloat16),
                          (((1,), (0,)), ((), ())),
                          preferred_element_type=f32)

    ones_row = jnp.ones((1, T), f32)
    counts = lax.dot_general(ones_row, mask, (((1,), (0,)), ((), ())),
                             preferred_element_type=f32)
    cnt_pad = jnp.ceil(counts / TILE) * TILE
    Us = (lax.broadcasted_iota(jnp.int32, (E, E), 0)
          < lax.broadcasted_iota(jnp.int32, (E, E), 1)).astype(f32)
    off = lax.dot_general(cnt_pad, Us, (((1,), (0,)), ((), ())),
                          preferred_element_type=f32)
    offpos = pos + off
    d1 = jnp.sum(oh1 * offpos, axis=1, keepdims=True)
    d2 = jnp.sum(oh2 * offpos, axis=1, keepdims=True)
    dst_ref[...] = jnp.concatenate([d1, d2], axis=1).astype(jnp.int32)

    ones_col = jnp.ones((T, 1), f32)
    countsc = lax.dot_general(mask, ones_col, (((0,), (0,)), ((), ())),
                              preferred_element_type=f32)
    cnt_padc = jnp.ceil(countsc / TILE) * TILE
    Ls8 = (lax.broadcasted_iota(jnp.int32, (E, E), 0)
           > lax.broadcasted_iota(jnp.int32, (E, E), 1)).astype(f32)
    offc = lax.dot_general(Ls8, cnt_padc, (((1,), (0,)), ((), ())),
                           preferred_element_type=f32)
    endc = offc + cnt_padc
    tstart = lax.broadcasted_iota(jnp.int32, (E, NT), 1).astype(f32) * TILE
    cmp = (endc <= tstart).astype(f32)
    raw = lax.dot_general(jnp.ones((1, E), f32), cmp, (((1,), (0,)), ((), ())),
                          preferred_element_type=f32)
    jj = lax.broadcasted_iota(jnp.int32, (1, NT), 1).astype(f32)
    is_sh = jj >= float(NRT)
    widx = jnp.where(is_sh, float(E), jnp.minimum(raw, float(E - 1)))
    live = jnp.where(is_sh, 1.0, (raw <= float(E - 1)).astype(f32))
    eot_ref[...] = jnp.concatenate([widx, live], axis=0).astype(jnp.int32)


def _ffn_body(eot_ref, xb_ref, wg_ref, wu_ref, wd_ref, out_ref):
    j = pl.program_id(0)

    @pl.when(eot_ref[1, j] == 1)
    def _():
        xb = xb_ref[...].astype(jnp.bfloat16)
        g = lax.dot_general(xb, wg_ref[0], (((1,), (1,)), ((), ())),
                            preferred_element_type=jnp.float32)
        u = lax.dot_general(xb, wu_ref[0], (((1,), (1,)), ((), ())),
                            preferred_element_type=jnp.float32)
        h = (g * jax.nn.sigmoid(g) * u).astype(jnp.bfloat16)
        out_ref[...] = lax.dot_general(h, wd_ref[0], (((1,), (1,)), ((), ())),
                                       preferred_element_type=jnp.float32)


def _worker_id():
    return lax.axis_index("s") * 2 + lax.axis_index("c")


def _dispatch_body(NP, CP, x_hbm, dst_hbm, xbuf_hbm, idx_v, tok_v, rows_v, sem):
    w = _worker_id()
    npairs = K * 2048 // NW
    for c in range(npairs // CP):
        base = w * npairs + c * CP
        pltpu.sync_copy(dst_hbm.at[pl.ds(base, CP)], idx_v)
        for jj in range(CP // LANES):
            tok_v[pl.ds(jj * LANES, LANES)] = (
                (lax.iota(jnp.int32, LANES) + (base + jj * LANES)) >> 1)
        pltpu.async_copy(x_hbm.at[tok_v], rows_v, sem).wait()
        pltpu.async_copy(rows_v, xbuf_hbm.at[idx_v], sem).wait()
    tb = w * CP
    pltpu.sync_copy(x_hbm.at[pl.ds(tb, CP)], rows_v)
    pltpu.sync_copy(rows_v, xbuf_hbm.at[pl.ds(NP + tb, CP)])


def _bcast_lane(vec, lane):
    idx = jnp.full((LANES,), lane, jnp.int32)
    return vec.at[idx].get(mode="promise_in_bounds")


def _combine_body(NP, D, obuf_hbm, dst_hbm, wts_hbm, out_hbm,
                  idx_v, wv_v, rows_v, shared_v, sem):
    w = _worker_id()
    TPW = 2048 // NW
    CT = 16
    for c in range(TPW // CT):
        t0 = w * TPW + c * CT
        pltpu.sync_copy(dst_hbm.at[pl.ds(2 * t0, 2 * CT)], idx_v)
        pltpu.sync_copy(wts_hbm.at[pl.ds(2 * t0, 2 * CT)], wv_v)
        gat = pltpu.async_copy(obuf_hbm.at[idx_v], rows_v, sem)
        pltpu.sync_copy(obuf_hbm.at[pl.ds(NP + t0, CT)], shared_v)
        gat.wait()
        wA = wv_v[pl.ds(0, LANES)]
        wB = wv_v[pl.ds(LANES, LANES)]
        w1s, w2s = [], []
        for i in range(CT):
            la, lb = 2 * i, 2 * i + 1
            w1s.append(_bcast_lane(wA if la < LANES else wB, la % LANES))
            w2s.append(_bcast_lane(wA if lb < LANES else wB, lb % LANES))

        def dchunk(k, carry):
            o = k * LANES
            for i in range(CT):
                shared_v[i, pl.ds(o, LANES)] = (
                    shared_v[i, pl.ds(o, LANES)]
                    + rows_v[2 * i, pl.ds(o, LANES)] * w1s[i]
                    + rows_v[2 * i + 1, pl.ds(o, LANES)] * w2s[i])
            return carry

        lax.fori_loop(0, D // LANES, dchunk, 0)
        pltpu.sync_copy(shared_v, out_hbm.at[pl.ds(t0, CT)])


def kernel(hidden_states, Wgate, Weg, Weu, Wed, Wsg, Wsu, Wsd):
    orig_shape = hidden_states.shape
    x = hidden_states.reshape(-1, orig_shape[-1])
    T, D = x.shape
    E, FF = Weg.shape[0], Weg.shape[1]
    NP = K * T + E * TILE
    NB = NP + T
    NT = NB // TILE
    NRT = NP // TILE

    dst, wts, eot = pl.pallas_call(
        functools.partial(_router_body, T, E, NT, NRT),
        out_shape=(
            jax.ShapeDtypeStruct((T, K), jnp.int32),
            jax.ShapeDtypeStruct((T, K), jnp.float32),
            jax.ShapeDtypeStruct((2, NT), jnp.int32),
        ),
    )(x, Wgate)
    dstv = dst.reshape(-1)
    wtsv = wts.reshape(-1)

    mesh = plsc.VectorSubcoreMesh(core_axis_name="c", subcore_axis_name="s",
                                  num_cores=2, num_subcores=16)
    CP = 64
    xbuf = pl.kernel(
        functools.partial(_dispatch_body, NP, CP),
        out_type=jax.ShapeDtypeStruct((NB, D), jnp.float32),
        mesh=mesh,
        scratch_types=[
            pltpu.VMEM((CP,), jnp.int32),
            pltpu.VMEM((CP,), jnp.int32),
            pltpu.VMEM((CP, D), jnp.float32),
            pltpu.SemaphoreType.DMA,
        ],
    )(x, dstv)

    Wg_all = jnp.concatenate([Weg, Wsg[None]], axis=0).astype(jnp.bfloat16)
    Wu_all = jnp.concatenate([Weu, Wsu[None]], axis=0).astype(jnp.bfloat16)
    Wd_all = jnp.concatenate([Wed, Wsd[None]], axis=0).astype(jnp.bfloat16)
    obuf = pl.pallas_call(
        _ffn_body,
        grid_spec=pltpu.PrefetchScalarGridSpec(
            num_scalar_prefetch=1,
            grid=(NT,),
            in_specs=[
                pl.BlockSpec((TILE, D), lambda j, eot: (j, 0)),
                pl.BlockSpec((1, FF, D), lambda j, eot: (eot[0, j], 0, 0)),
                pl.BlockSpec((1, FF, D), lambda j, eot: (eot[0, j], 0, 0)),
                pl.BlockSpec((1, D, FF), lambda j, eot: (eot[0, j], 0, 0)),
            ],
            out_specs=pl.BlockSpec((TILE, D), lambda j, eot: (j, 0)),
        ),
        out_shape=jax.ShapeDtypeStruct((NB, D), jnp.float32),
    )(eot, xbuf, Wg_all, Wu_all, Wd_all)

    CT = 16
    out = pl.kernel(
        functools.partial(_combine_body, NP, D),
        out_type=jax.ShapeDtypeStruct((T, D), jnp.float32),
        mesh=mesh,
        scratch_types=[
            pltpu.VMEM((2 * CT,), jnp.int32),
            pltpu.VMEM((2 * CT,), jnp.float32),
            pltpu.VMEM((2 * CT, D), jnp.float32),
            pltpu.VMEM((CT, D), jnp.float32),
            pltpu.SemaphoreType.DMA,
        ],
    )(obuf, dstv, wtsv)

    return out.reshape(orig_shape)
